# Initial kernel scaffold; baseline (speedup 1.0000x reference)
import jax
import jax.numpy as jnp
from jax.experimental import pallas as pl


def kernel(node_emb, edge_emb, feature_emb, edge_index, P_src, b_P_src, P_dst, b_P_dst, Q_src_self, b_Q_src_self, Q_src_neigh, b_Q_src_neigh, Q_dst_self, b_Q_dst_self, Q_dst_neigh, b_Q_dst_neigh, W, b_W):
    raise NotImplementedError("write your pallas kernel here")


# speedup vs baseline: 1.6030x; 1.6030x over previous
import functools

import jax
import jax.numpy as jnp
from jax import lax
from jax.experimental import pallas as pl
from jax.experimental.pallas import tpu as pltpu
from jax.experimental.pallas import tpu_sc as plsc

N_NODES = 10000
N_EDGES = 320000
D = 128
DH = 64
DE = 16

AB_CHUNK = 80
AB_NCHUNK = 250
AB_SEG = 50
AB_NSEG = AB_NCHUNK // AB_SEG
AB_PER_TILE = AB_CHUNK * AB_NCHUNK
AB_ZROWS = 64
NPAD = 10240
STRIPE = NPAD // 16
ZROWS = 128

C_CHUNK = 80
C_NCHUNK = 125
C_PER_TILE = C_CHUNK * C_NCHUNK


def _tc_nodes_body(feat_ref, node_ref, psf_ref, pdf_ref, t_ref):
    t_ref[0] = jnp.dot(feat_ref[...], psf_ref[...],
                       preferred_element_type=jnp.float32)
    t_ref[1] = jnp.dot(node_ref[...], pdf_ref[...],
                       preferred_element_type=jnp.float32)


def _tc_nodes(feature_emb, node_emb, psf, pdf):
    nb = 10
    rows = N_NODES // nb
    return pl.pallas_call(
        _tc_nodes_body,
        grid=(nb,),
        in_specs=[
            pl.BlockSpec((rows, D), lambda i: (i, 0)),
            pl.BlockSpec((rows, D), lambda i: (i, 0)),
            pl.BlockSpec((D, D), lambda i: (0, 0)),
            pl.BlockSpec((D, D), lambda i: (0, 0)),
        ],
        out_specs=pl.BlockSpec((2, rows, D), lambda i: (0, i, 0)),
        out_shape=jax.ShapeDtypeStruct((2, N_NODES, D), jnp.float32),
    )(feature_emb, node_emb, psf, pdf)


def _tc_edges_body(ee_ref, pse_ref, pde_ref, we_ref, bps_ref, bpd_ref, bw_ref,
                   e_ref, e3_ref):
    ee = ee_ref[...]
    e_ref[0] = jnp.dot(ee, pse_ref[...],
                       preferred_element_type=jnp.float32) + bps_ref[...]
    e_ref[1] = jnp.dot(ee, pde_ref[...],
                       preferred_element_type=jnp.float32) + bpd_ref[...]
    e3_ref[...] = jnp.dot(ee, we_ref[...],
                          preferred_element_type=jnp.float32) + bw_ref[...]


def _tc_edges(edge_emb, pse, pde, we, bps, bpd, bw):
    eb = 2000
    nb = N_EDGES // eb
    return pl.pallas_call(
        _tc_edges_body,
        grid=(nb,),
        in_specs=[
            pl.BlockSpec((eb, DE), lambda i: (i, 0)),
            pl.BlockSpec((DE, D), lambda i: (0, 0)),
            pl.BlockSpec((DE, D), lambda i: (0, 0)),
            pl.BlockSpec((DE, DE), lambda i: (0, 0)),
            pl.BlockSpec((1, D), lambda i: (0, 0)),
            pl.BlockSpec((1, D), lambda i: (0, 0)),
            pl.BlockSpec((1, DE), lambda i: (0, 0)),
        ],
        out_specs=[
            pl.BlockSpec((2, eb, D), lambda i: (0, i, 0)),
            pl.BlockSpec((eb, DE), lambda i: (i, 0)),
        ],
        out_shape=[
            jax.ShapeDtypeStruct((2, N_EDGES, D), jnp.float32),
            jax.ShapeDtypeStruct((N_EDGES, DE), jnp.float32),
        ],
    )(edge_emb, pse, pde, we, bps, bpd, bw)


def _sc_ab_body(tab_hbm, est_hbm, gidx_hbm, sidx_hbm,
                acc0_hbm, acc1_hbm,
                gidx_v, sidx_v, gbuf, ebuf, y0buf, y1buf,
                zbuf, acc0_s, acc1_s):
    c = lax.axis_index("c")
    s = lax.axis_index("s")

    zero16 = jnp.zeros((16,), jnp.float32)

    def zfill(i, _):
        for v in range(DH // 16):
            zbuf[i, pl.ds(v * 16, 16)] = zero16
        return 0
    lax.fori_loop(0, AB_ZROWS, zfill, 0)

    def zero_stripe(k, _):
        r0 = s * STRIPE + k * AB_ZROWS
        pltpu.sync_copy(zbuf, acc0_s.at[pl.ds(r0, AB_ZROWS)])
        pltpu.sync_copy(zbuf, acc1_s.at[pl.ds(r0, AB_ZROWS)])
        return 0
    lax.fori_loop(0, STRIPE // AB_ZROWS, zero_stripe, 0)
    plsc.subcore_barrier()

    ebase = c * N_EDGES + s * AB_PER_TILE

    def seg(g, _):
        pltpu.sync_copy(gidx_hbm.at[c, s, g], gidx_v)
        pltpu.sync_copy(sidx_hbm.at[c, s, g], sidx_v)

        def chunk(j, _):
            pltpu.sync_copy(tab_hbm.at[gidx_v.at[j]], gbuf)
            pltpu.sync_copy(
                est_hbm.at[pl.ds(ebase + (g * AB_SEG + j) * AB_CHUNK,
                                 AB_CHUNK)], ebuf)

            def edge(i, _):
                for v in range(4):
                    sl = pl.ds(v * 16, 16)
                    t = gbuf[i, sl] + ebuf[i, sl]
                    y0buf[i, sl] = jnp.maximum(t, t * 0.01)
                for v in range(4):
                    sl = pl.ds(DH + v * 16, 16)
                    t = gbuf[i, sl] + ebuf[i, sl]
                    y1buf[i, pl.ds(v * 16, 16)] = jnp.maximum(t, t * 0.01)
                return 0
            lax.fori_loop(0, AB_CHUNK, edge, 0)

            pltpu.sync_copy(y0buf, acc0_s.at[sidx_v.at[j]], add=True)
            pltpu.sync_copy(y1buf, acc1_s.at[sidx_v.at[j]], add=True)
            return 0
        lax.fori_loop(0, AB_SEG, chunk, 0)
        return 0
    lax.fori_loop(0, AB_NSEG, seg, 0)
    plsc.subcore_barrier()

    def flush(k, _):
        r0 = s * STRIPE + k * AB_ZROWS
        pltpu.sync_copy(acc0_s.at[pl.ds(r0, AB_ZROWS)], zbuf)
        pltpu.sync_copy(zbuf, acc0_hbm.at[pl.ds(c * NPAD + r0, AB_ZROWS)])
        pltpu.sync_copy(acc1_s.at[pl.ds(r0, AB_ZROWS)], zbuf)
        pltpu.sync_copy(zbuf, acc1_hbm.at[pl.ds(c * NPAD + r0, AB_ZROWS)])
        return 0
    lax.fori_loop(0, STRIPE // AB_ZROWS, flush, 0)


def _sc_ab(tab, est, gidx, sidx):
    mesh = plsc.VectorSubcoreMesh(core_axis_name="c", subcore_axis_name="s")
    f = functools.partial(
        pl.kernel, _sc_ab_body, mesh=mesh,
        compiler_params=pltpu.CompilerParams(use_tc_tiling_on_sc=False),
        out_type=[
            jax.ShapeDtypeStruct((2 * NPAD, DH), jnp.float32),
            jax.ShapeDtypeStruct((2 * NPAD, DH), jnp.float32),
        ],
        scratch_types=[
            pltpu.VMEM((AB_SEG, AB_CHUNK), jnp.int32),
            pltpu.VMEM((AB_SEG, AB_CHUNK), jnp.int32),
            pltpu.VMEM((AB_CHUNK, D), jnp.float32),
            pltpu.VMEM((AB_CHUNK, D), jnp.float32),
            pltpu.VMEM((AB_CHUNK, DH), jnp.float32),
            pltpu.VMEM((AB_CHUNK, DH), jnp.float32),
            pltpu.VMEM((AB_ZROWS, DH), jnp.float32),
            pltpu.VMEM_SHARED((NPAD, DH), jnp.float32),
            pltpu.VMEM_SHARED((NPAD, DH), jnp.float32),
        ],
    )()
    return f(tab, est, gidx, sidx)


def _sc_cnt_body(sidx_hbm, cnt_hbm, sidx_v, ones_v, zbuf16, cnt_s):
    c = lax.axis_index("c")
    s = lax.axis_index("s")

    ones16 = jnp.full((16,), 1.0, jnp.float32)
    zero16 = jnp.zeros((16,), jnp.float32)

    def fill(i, _):
        ones_v[i] = ones16
        return 0
    lax.fori_loop(0, AB_CHUNK, fill, 0)

    def zfill(i, _):
        zbuf16[i] = zero16
        return 0
    lax.fori_loop(0, ZROWS, zfill, 0)

    def zero_stripe(k, _):
        r0 = s * STRIPE + k * ZROWS
        pltpu.sync_copy(zbuf16, cnt_s.at[pl.ds(r0, ZROWS)])
        return 0
    lax.fori_loop(0, STRIPE // ZROWS, zero_stripe, 0)
    plsc.subcore_barrier()

    def seg(g, _):
        pltpu.sync_copy(sidx_hbm.at[c, s, g], sidx_v)

        def chunk(j, _):
            pltpu.sync_copy(ones_v, cnt_s.at[sidx_v.at[j]], add=True)
            return 0
        lax.fori_loop(0, AB_SEG, chunk, 0)
        return 0
    lax.fori_loop(0, AB_NSEG, seg, 0)
    plsc.subcore_barrier()

    def flush(k, _):
        r0 = s * STRIPE + k * ZROWS
        pltpu.sync_copy(cnt_s.at[pl.ds(r0, ZROWS)], zbuf16)
        pltpu.sync_copy(zbuf16, cnt_hbm.at[pl.ds(c * NPAD + r0, ZROWS)])
        return 0
    lax.fori_loop(0, STRIPE // ZROWS, flush, 0)


def _sc_cnt(sidx):
    mesh = plsc.VectorSubcoreMesh(core_axis_name="c", subcore_axis_name="s")
    f = functools.partial(
        pl.kernel, _sc_cnt_body, mesh=mesh,
        compiler_params=pltpu.CompilerParams(use_tc_tiling_on_sc=False),
        out_type=jax.ShapeDtypeStruct((2 * NPAD, 16), jnp.float32),
        scratch_types=[
            pltpu.VMEM((AB_SEG, AB_CHUNK), jnp.int32),
            pltpu.VMEM((AB_CHUNK, 16), jnp.float32),
            pltpu.VMEM((ZROWS, 16), jnp.float32),
            pltpu.VMEM_SHARED((NPAD, 16), jnp.float32),
        ],
    )()
    return f(sidx)


def _tc_stage2_body(pa0_ref, pa1_ref, pb0_ref, pb1_ref, ca_ref, cb_ref,
                    ne_ref, fe_ref,
                    qss_ref, qsn_ref, qds_ref, qdn_ref, wn_ref, wf_ref,
                    bn_ref, bf_ref, no_ref, fo_ref, nw_ref, fw_ref):
    inv_a = 1.0 / jnp.clip(ca_ref[:, 0:1], 1.0, None)
    inv_b = 1.0 / jnp.clip(cb_ref[:, 0:1], 1.0, None)
    msga = jnp.concatenate([pa0_ref[...], pa1_ref[...]], axis=1) * inv_a
    msgb = jnp.concatenate([pb0_ref[...], pb1_ref[...]], axis=1) * inv_b
    node_out = (jnp.dot(ne_ref[...], qss_ref[...],
                        preferred_element_type=jnp.float32)
                + jnp.dot(msga, qsn_ref[...],
                          preferred_element_type=jnp.float32)
                + bn_ref[...])
    feat_out = (jnp.dot(fe_ref[...], qds_ref[...],
                        preferred_element_type=jnp.float32)
                + jnp.dot(msgb, qdn_ref[...],
                          preferred_element_type=jnp.float32)
                + bf_ref[...])
    no_ref[...] = node_out
    fo_ref[...] = feat_out
    nw_ref[...] = jnp.dot(node_out, wn_ref[...],
                          preferred_element_type=jnp.float32)
    fw_ref[...] = jnp.dot(feat_out, wf_ref[...],
                          preferred_element_type=jnp.float32)


def _tc_stage2(acc0, acc1, cnt, node_emb, feature_emb, qss, qsn, qds, qdn,
               wn, wf, bn, bf):
    rows = 80
    nb = N_NODES // rows
    pb = NPAD // rows
    return pl.pallas_call(
        _tc_stage2_body,
        grid=(nb,),
        in_specs=[
            pl.BlockSpec((rows, DH), lambda i: (i, 0)),
            pl.BlockSpec((rows, DH), lambda i: (i, 0)),
            pl.BlockSpec((rows, DH), lambda i: (i + pb, 0)),
            pl.BlockSpec((rows, DH), lambda i: (i + pb, 0)),
            pl.BlockSpec((rows, 16), lambda i: (i, 0)),
            pl.BlockSpec((rows, 16), lambda i: (i + pb, 0)),
            pl.BlockSpec((rows, D), lambda i: (i, 0)),
            pl.BlockSpec((rows, D), lambda i: (i, 0)),
            pl.BlockSpec((D, D), lambda i: (0, 0)),
            pl.BlockSpec((D, D), lambda i: (0, 0)),
            pl.BlockSpec((D, D), lambda i: (0, 0)),
            pl.BlockSpec((D, D), lambda i: (0, 0)),
            pl.BlockSpec((D, DE), lambda i: (0, 0)),
            pl.BlockSpec((D, DE), lambda i: (0, 0)),
            pl.BlockSpec((1, D), lambda i: (0, 0)),
            pl.BlockSpec((1, D), lambda i: (0, 0)),
        ],
        out_specs=[
            pl.BlockSpec((rows, D), lambda i: (i, 0)),
            pl.BlockSpec((rows, D), lambda i: (i, 0)),
            pl.BlockSpec((rows, DE), lambda i: (i, 0)),
            pl.BlockSpec((rows, DE), lambda i: (i, 0)),
        ],
        out_shape=[
            jax.ShapeDtypeStruct((N_NODES, D), jnp.float32),
            jax.ShapeDtypeStruct((N_NODES, D), jnp.float32),
            jax.ShapeDtypeStruct((N_NODES, DE), jnp.float32),
            jax.ShapeDtypeStruct((N_NODES, DE), jnp.float32),
        ],
    )(acc0, acc1, acc0, acc1, cnt, cnt, node_emb, feature_emb,
      qss, qsn, qds, qdn, wn, wf, bn, bf)


def _sc_c_body(no_hbm, fo_hbm, e3_hbm, idx_hbm, out_hbm,
               sidx_v, didx_v, nbuf, fbuf, ebuf, obuf):
    c = lax.axis_index("c")
    s = lax.axis_index("s")
    wid = s * 2 + c

    pltpu.sync_copy(idx_hbm.at[wid], sidx_v)
    pltpu.sync_copy(idx_hbm.at[32 + wid], didx_v)
    ebase = wid * C_PER_TILE

    def chunk(j, _):
        pltpu.sync_copy(no_hbm.at[sidx_v.at[j]], nbuf)
        pltpu.sync_copy(fo_hbm.at[didx_v.at[j]], fbuf)
        pltpu.sync_copy(e3_hbm.at[pl.ds(ebase + j * C_CHUNK, C_CHUNK)], ebuf)

        def edge(i, _):
            obuf[i] = nbuf[i] + fbuf[i] + ebuf[i]
            return 0
        lax.fori_loop(0, C_CHUNK, edge, 0)

        pltpu.sync_copy(obuf, out_hbm.at[pl.ds(ebase + j * C_CHUNK, C_CHUNK)])
        return 0
    lax.fori_loop(0, C_NCHUNK, chunk, 0)


def _sc_c(no, fo, e3, idx):
    mesh = plsc.VectorSubcoreMesh(core_axis_name="c", subcore_axis_name="s")
    f = functools.partial(
        pl.kernel, _sc_c_body, mesh=mesh,
        compiler_params=pltpu.CompilerParams(use_tc_tiling_on_sc=False),
        out_type=jax.ShapeDtypeStruct((N_EDGES, DE), jnp.float32),
        scratch_types=[
            pltpu.VMEM((C_NCHUNK, C_CHUNK), jnp.int32),
            pltpu.VMEM((C_NCHUNK, C_CHUNK), jnp.int32),
            pltpu.VMEM((C_CHUNK, DE), jnp.float32),
            pltpu.VMEM((C_CHUNK, DE), jnp.float32),
            pltpu.VMEM((C_CHUNK, DE), jnp.float32),
            pltpu.VMEM((C_CHUNK, DE), jnp.float32),
        ],
    )()
    return f(no, fo, e3, idx)


def kernel(node_emb, edge_emb, feature_emb, edge_index,
           P_src, b_P_src, P_dst, b_P_dst,
           Q_src_self, b_Q_src_self, Q_src_neigh, b_Q_src_neigh,
           Q_dst_self, b_Q_dst_self, Q_dst_neigh, b_Q_dst_neigh,
           W, b_W):
    psf, pse = P_src[:D], P_src[D:]
    pdf, pde = P_dst[:D], P_dst[D:]
    we, wn, wf = W[:DE], W[DE:DE + D], W[DE + D:]
    bps = b_P_src.reshape(1, D)
    bpd = b_P_dst.reshape(1, D)
    bw = b_W.reshape(1, DE)
    bn = (b_Q_src_self + b_Q_src_neigh).reshape(1, D)
    bf = (b_Q_dst_self + b_Q_dst_neigh).reshape(1, D)

    src = edge_index[0]
    dst = edge_index[1]
    gidx = jnp.stack([dst, src + N_NODES]).reshape(
        2, 16, AB_NSEG, AB_SEG, AB_CHUNK)
    sidx = edge_index.reshape(2, 16, AB_NSEG, AB_SEG, AB_CHUNK)
    idx_c = edge_index.reshape(64, C_NCHUNK, C_CHUNK)

    tab = _tc_nodes(feature_emb, node_emb, psf, pdf)
    est, e3 = _tc_edges(edge_emb, pse, pde, we, bps, bpd, bw)

    cnt = _sc_cnt(sidx)
    acc0, acc1 = _sc_ab(tab.reshape(2 * N_NODES, D),
                        est.reshape(2 * N_EDGES, D), gidx, sidx)

    node_out, feat_out, no_w, fo_w = _tc_stage2(
        acc0, acc1, cnt, node_emb, feature_emb,
        Q_src_self, Q_src_neigh, Q_dst_self, Q_dst_neigh,
        wn, wf, bn, bf)

    edge_out = _sc_c(no_w, fo_w, e3, idx_c)
    return (node_out, edge_out, feat_out)


# speedup vs baseline: 3.2240x; 2.0113x over previous
import functools

import jax
import jax.numpy as jnp
from jax import lax
from jax.experimental import pallas as pl
from jax.experimental.pallas import tpu as pltpu
from jax.experimental.pallas import tpu_sc as plsc

N_NODES = 10000
N_EDGES = 320000
D = 128
DH = 64
DE = 16

AB_CHUNK = 40
AB_NCHUNK = 500
AB_SEG = 100
AB_NSEG = AB_NCHUNK // AB_SEG
AB_PER_TILE = AB_CHUNK * AB_NCHUNK
AB_ZROWS = 64
NPAD = 10240
STRIPE = NPAD // 16
ZROWS = 128

C_CHUNK = 40
C_NCHUNK = 250
C_PER_TILE = C_CHUNK * C_NCHUNK


def _tc_nodes_body(feat_ref, node_ref, psf_ref, pdf_ref, t0_ref, t1_ref):
    f1 = jnp.dot(feat_ref[...], psf_ref[...],
                 preferred_element_type=jnp.float32)
    n1 = jnp.dot(node_ref[...], pdf_ref[...],
                 preferred_element_type=jnp.float32)
    t0_ref[0] = f1[:, :DH]
    t0_ref[1] = n1[:, :DH]
    t1_ref[0] = f1[:, DH:]
    t1_ref[1] = n1[:, DH:]


def _tc_nodes(feature_emb, node_emb, psf, pdf):
    nb = 10
    rows = N_NODES // nb
    return pl.pallas_call(
        _tc_nodes_body,
        grid=(nb,),
        in_specs=[
            pl.BlockSpec((rows, D), lambda i: (i, 0)),
            pl.BlockSpec((rows, D), lambda i: (i, 0)),
            pl.BlockSpec((D, D), lambda i: (0, 0)),
            pl.BlockSpec((D, D), lambda i: (0, 0)),
        ],
        out_specs=[
            pl.BlockSpec((2, rows, DH), lambda i: (0, i, 0)),
            pl.BlockSpec((2, rows, DH), lambda i: (0, i, 0)),
        ],
        out_shape=[
            jax.ShapeDtypeStruct((2, N_NODES, DH), jnp.float32),
            jax.ShapeDtypeStruct((2, N_NODES, DH), jnp.float32),
        ],
    )(feature_emb, node_emb, psf, pdf)


def _tc_edges_body(ee_ref, p_ref, b_ref, est_ref):
    est_ref[...] = jnp.dot(ee_ref[...], p_ref[...],
                           preferred_element_type=jnp.float32) + b_ref[0:1]


def _tc_edges(edge_emb, pse, pde, bps, bpd):
    eb = 2000
    nb = N_EDGES // eb
    p2 = jnp.concatenate([pse, pde], axis=0)
    b2 = jnp.concatenate([jnp.broadcast_to(bps, (8, D)),
                          jnp.broadcast_to(bpd, (8, D))], axis=0)
    return pl.pallas_call(
        _tc_edges_body,
        grid=(2, nb),
        in_specs=[
            pl.BlockSpec((eb, DE), lambda p, i: (i, 0)),
            pl.BlockSpec((DE, D), lambda p, i: (p, 0)),
            pl.BlockSpec((8, D), lambda p, i: (p, 0)),
        ],
        out_specs=pl.BlockSpec((eb, D), lambda p, i: (p * nb + i, 0)),
        out_shape=jax.ShapeDtypeStruct((2 * N_EDGES, D), jnp.float32),
    )(edge_emb, p2, b2)


def _tc_e3_body(ee_ref, we_ref, bw_ref, e3_ref):
    e3_ref[...] = jnp.dot(ee_ref[...], we_ref[...],
                          preferred_element_type=jnp.float32) + bw_ref[...]


def _tc_e3(edge_emb, we, bw):
    eb = 2000
    nb = N_EDGES // eb
    return pl.pallas_call(
        _tc_e3_body,
        grid=(nb,),
        in_specs=[
            pl.BlockSpec((eb, DE), lambda i: (i, 0)),
            pl.BlockSpec((DE, DE), lambda i: (0, 0)),
            pl.BlockSpec((1, DE), lambda i: (0, 0)),
        ],
        out_specs=pl.BlockSpec((eb, DE), lambda i: (i, 0)),
        out_shape=jax.ShapeDtypeStruct((N_EDGES, DE), jnp.float32),
    )(edge_emb, we, bw)


def _sc_ab_body(tab0_hbm, tab1_hbm, est_hbm, gidxa_hbm, gidxb_hbm, sidx_hbm,
                acc0_hbm, acc1_hbm, cnt_hbm,
                gidx_v, sidx_v, g0, g1, g2, g3, e0, e1, e2, e3,
                ones_v, zbuf, cbuf, acc0_s, acc1_s, cnt_s,
                gs0, gs1, gs2, gs3, es0, es1, es2, es3,
                ss0, ss1, ss2, ss3, cs0, cs1, cs2, cs3):
    c = lax.axis_index("c")
    s = lax.axis_index("s")
    gbufs = (g0, g1, g2, g3)
    ebufs = (e0, e1, e2, e3)
    gsems = (gs0, gs1, gs2, gs3)
    esems = (es0, es1, es2, es3)
    ssems = (ss0, ss1, ss2, ss3)
    csems = (cs0, cs1, cs2, cs3)
    tabs = (tab0_hbm, tab1_hbm)
    accs = (acc0_s, acc1_s)

    zero16 = jnp.zeros((16,), jnp.float32)
    ones16 = jnp.full((16,), 1.0, jnp.float32)

    def ofill(i, _):
        ones_v[i] = ones16
        return 0
    lax.fori_loop(0, AB_CHUNK, ofill, 0)

    def zfill(i, _):
        for v in range(DH // 16):
            zbuf[i, pl.ds(v * 16, 16)] = zero16
        cbuf[i] = zero16
        return 0
    lax.fori_loop(0, AB_ZROWS, zfill, 0)

    def zero_stripe(k, _):
        r0 = s * STRIPE + k * AB_ZROWS
        pltpu.sync_copy(zbuf, acc0_s.at[pl.ds(r0, AB_ZROWS)])
        pltpu.sync_copy(zbuf, acc1_s.at[pl.ds(r0, AB_ZROWS)])
        pltpu.sync_copy(cbuf, cnt_s.at[pl.ds(r0, AB_ZROWS)])
        return 0
    lax.fori_loop(0, STRIPE // AB_ZROWS, zero_stripe, 0)
    plsc.subcore_barrier()

    ebase = c * N_EDGES + s * AB_PER_TILE

    def start_in(r, g, j, k):
        pltpu.async_copy(tabs[r].at[gidx_v.at[j]], gbufs[k], gsems[k])
        pltpu.async_copy(
            est_hbm.at[pl.ds(ebase + (g * AB_SEG + j) * AB_CHUNK, AB_CHUNK),
                       pl.ds(r * DH, DH)],
            ebufs[k], esems[k])

    def wait_in(r, j, k):
        pltpu.make_async_copy(tabs[r].at[gidx_v.at[j]], gbufs[k],
                              gsems[k]).wait()
        pltpu.make_async_copy(
            est_hbm.at[pl.ds(ebase, AB_CHUNK), pl.ds(r * DH, DH)],
            ebufs[k], esems[k]).wait()

    def wait_scat(r, j, k):
        pltpu.make_async_copy(gbufs[k], accs[r].at[sidx_v.at[j]],
                              ssems[k]).wait()
        if r == 0:
            pltpu.make_async_copy(ones_v, cnt_s.at[sidx_v.at[j]],
                                  csems[k]).wait()

    def seg(g, _):
        @pl.when(c == 0)
        def _():
            pltpu.sync_copy(gidxa_hbm.at[s, g], gidx_v)

        @pl.when(c == 1)
        def _():
            pltpu.sync_copy(gidxb_hbm.at[s, g], gidx_v)

        pltpu.sync_copy(sidx_hbm.at[c, s, g], sidx_v)
        for r in range(2):
            acc_s = accs[r]
            for k in range(4):
                start_in(r, g, k, k)

            def step(jj, _):
                for k in range(4):
                    j = 4 * jj + k
                    wait_in(r, j, k)
                    gbuf = gbufs[k]
                    ebuf = ebufs[k]

                    def edge(i, _):
                        for v in range(DH // 16):
                            sl = pl.ds(v * 16, 16)
                            t = gbuf[i, sl] + ebuf[i, sl]
                            gbuf[i, sl] = jnp.maximum(t, t * 0.01)
                        return 0
                    lax.fori_loop(0, AB_CHUNK, edge, 0)

                    pltpu.async_copy(gbuf, acc_s.at[sidx_v.at[j]], ssems[k],
                                     add=True)
                    if r == 0:
                        pltpu.async_copy(ones_v, cnt_s.at[sidx_v.at[j]],
                                         csems[k], add=True)

                    @pl.when(jj < AB_SEG // 4 - 1)
                    def _():
                        wait_scat(r, j, k)
                        start_in(r, g, j + 4, k)
                return 0
            lax.fori_loop(0, AB_SEG // 4, step, 0)

            for k in range(4):
                wait_scat(r, AB_SEG - 4 + k, k)
        return 0
    lax.fori_loop(0, AB_NSEG, seg, 0)
    plsc.subcore_barrier()

    def flush(k, _):
        r0 = s * STRIPE + k * AB_ZROWS
        pltpu.sync_copy(acc0_s.at[pl.ds(r0, AB_ZROWS)], zbuf)
        pltpu.sync_copy(zbuf, acc0_hbm.at[pl.ds(c * NPAD + r0, AB_ZROWS)])
        pltpu.sync_copy(acc1_s.at[pl.ds(r0, AB_ZROWS)], zbuf)
        pltpu.sync_copy(zbuf, acc1_hbm.at[pl.ds(c * NPAD + r0, AB_ZROWS)])
        pltpu.sync_copy(cnt_s.at[pl.ds(r0, AB_ZROWS)], cbuf)
        pltpu.sync_copy(cbuf, cnt_hbm.at[pl.ds(c * NPAD + r0, AB_ZROWS)])
        return 0
    lax.fori_loop(0, STRIPE // AB_ZROWS, flush, 0)


def _sc_ab(tab0, tab1, est, gidxa, gidxb, sidx):
    mesh = plsc.VectorSubcoreMesh(core_axis_name="c", subcore_axis_name="s")
    f = functools.partial(
        pl.kernel, _sc_ab_body, mesh=mesh,
        compiler_params=pltpu.CompilerParams(use_tc_tiling_on_sc=False),
        out_type=[
            jax.ShapeDtypeStruct((2 * NPAD, DH), jnp.float32),
            jax.ShapeDtypeStruct((2 * NPAD, DH), jnp.float32),
            jax.ShapeDtypeStruct((2 * NPAD, 16), jnp.float32),
        ],
        scratch_types=(
            [pltpu.VMEM((AB_SEG, AB_CHUNK), jnp.int32)] * 2
            + [pltpu.VMEM((AB_CHUNK, DH), jnp.float32)] * 8
            + [pltpu.VMEM((AB_CHUNK, 16), jnp.float32)]
            + [pltpu.VMEM((AB_ZROWS, DH), jnp.float32)]
            + [pltpu.VMEM((AB_ZROWS, 16), jnp.float32)]
            + [pltpu.VMEM_SHARED((NPAD, DH), jnp.float32)] * 2
            + [pltpu.VMEM_SHARED((NPAD, 16), jnp.float32)]
            + [pltpu.SemaphoreType.DMA] * 16
        ),
    )()
    return f(tab0, tab1, est, gidxa, gidxb, sidx)


def _tc_stage2_body(pa0_ref, pa1_ref, pb0_ref, pb1_ref, ca_ref, cb_ref,
                    ne_ref, fe_ref,
                    qss_ref, qsn_ref, qds_ref, qdn_ref, wn_ref, wf_ref,
                    bn_ref, bf_ref, no_ref, fo_ref, nw_ref, fw_ref):
    inv_a = 1.0 / jnp.clip(ca_ref[:, 0:1], 1.0, None)
    inv_b = 1.0 / jnp.clip(cb_ref[:, 0:1], 1.0, None)
    msga = jnp.concatenate([pa0_ref[...], pa1_ref[...]], axis=1) * inv_a
    msgb = jnp.concatenate([pb0_ref[...], pb1_ref[...]], axis=1) * inv_b
    node_out = (jnp.dot(ne_ref[...], qss_ref[...],
                        preferred_element_type=jnp.float32)
                + jnp.dot(msga, qsn_ref[...],
                          preferred_element_type=jnp.float32)
                + bn_ref[...])
    feat_out = (jnp.dot(fe_ref[...], qds_ref[...],
                        preferred_element_type=jnp.float32)
                + jnp.dot(msgb, qdn_ref[...],
                          preferred_element_type=jnp.float32)
                + bf_ref[...])
    no_ref[...] = node_out
    fo_ref[...] = feat_out
    nw_ref[...] = jnp.dot(node_out, wn_ref[...],
                          preferred_element_type=jnp.float32)
    fw_ref[...] = jnp.dot(feat_out, wf_ref[...],
                          preferred_element_type=jnp.float32)


def _tc_stage2(acc0, acc1, cnt, node_emb, feature_emb, qss, qsn, qds, qdn,
               wn, wf, bn, bf):
    rows = 80
    nb = N_NODES // rows
    pb = NPAD // rows
    return pl.pallas_call(
        _tc_stage2_body,
        grid=(nb,),
        in_specs=[
            pl.BlockSpec((rows, DH), lambda i: (i, 0)),
            pl.BlockSpec((rows, DH), lambda i: (i, 0)),
            pl.BlockSpec((rows, DH), lambda i: (i + pb, 0)),
            pl.BlockSpec((rows, DH), lambda i: (i + pb, 0)),
            pl.BlockSpec((rows, 16), lambda i: (i, 0)),
            pl.BlockSpec((rows, 16), lambda i: (i + pb, 0)),
            pl.BlockSpec((rows, D), lambda i: (i, 0)),
            pl.BlockSpec((rows, D), lambda i: (i, 0)),
            pl.BlockSpec((D, D), lambda i: (0, 0)),
            pl.BlockSpec((D, D), lambda i: (0, 0)),
            pl.BlockSpec((D, D), lambda i: (0, 0)),
            pl.BlockSpec((D, D), lambda i: (0, 0)),
            pl.BlockSpec((D, DE), lambda i: (0, 0)),
            pl.BlockSpec((D, DE), lambda i: (0, 0)),
            pl.BlockSpec((1, D), lambda i: (0, 0)),
            pl.BlockSpec((1, D), lambda i: (0, 0)),
        ],
        out_specs=[
            pl.BlockSpec((rows, D), lambda i: (i, 0)),
            pl.BlockSpec((rows, D), lambda i: (i, 0)),
            pl.BlockSpec((rows, DE), lambda i: (i, 0)),
            pl.BlockSpec((rows, DE), lambda i: (i, 0)),
        ],
        out_shape=[
            jax.ShapeDtypeStruct((N_NODES, D), jnp.float32),
            jax.ShapeDtypeStruct((N_NODES, D), jnp.float32),
            jax.ShapeDtypeStruct((N_NODES, DE), jnp.float32),
            jax.ShapeDtypeStruct((N_NODES, DE), jnp.float32),
        ],
    )(acc0, acc1, acc0, acc1, cnt, cnt, node_emb, feature_emb,
      qss, qsn, qds, qdn, wn, wf, bn, bf)


def _sc_c_body(no_hbm, fo_hbm, e3_hbm, idx_hbm, out_hbm,
               sidx_v, didx_v, nb0, nb1, fb0, fb1, eb0, eb1, ob0, ob1,
               nsem0, nsem1, fsem0, fsem1, esem0, esem1, osem0, osem1):
    c = lax.axis_index("c")
    s = lax.axis_index("s")
    wid = s * 2 + c
    nbufs = (nb0, nb1)
    fbufs = (fb0, fb1)
    ebufs = (eb0, eb1)
    obufs = (ob0, ob1)
    nsems = (nsem0, nsem1)
    fsems = (fsem0, fsem1)
    esems = (esem0, esem1)
    osems = (osem0, osem1)

    pltpu.sync_copy(idx_hbm.at[wid], sidx_v)
    pltpu.sync_copy(idx_hbm.at[32 + wid], didx_v)
    ebase = wid * C_PER_TILE

    def start_in(j, b):
        pltpu.async_copy(no_hbm.at[sidx_v.at[j]], nbufs[b], nsems[b])
        pltpu.async_copy(fo_hbm.at[didx_v.at[j]], fbufs[b], fsems[b])
        pltpu.async_copy(e3_hbm.at[pl.ds(ebase + j * C_CHUNK, C_CHUNK)],
                         ebufs[b], esems[b])

    def wait_in(j, b):
        pltpu.make_async_copy(no_hbm.at[sidx_v.at[j]], nbufs[b],
                              nsems[b]).wait()
        pltpu.make_async_copy(fo_hbm.at[didx_v.at[j]], fbufs[b],
                              fsems[b]).wait()
        pltpu.make_async_copy(e3_hbm.at[pl.ds(ebase, C_CHUNK)], ebufs[b],
                              esems[b]).wait()

    def wait_out(b):
        pltpu.make_async_copy(obufs[b],
                              out_hbm.at[pl.ds(ebase, C_CHUNK)],
                              osems[b]).wait()

    for b in range(2):
        start_in(b, b)

    def step(jj, _):
        for b in range(2):
            j = 2 * jj + b
            wait_in(j, b)

            @pl.when(jj > 0)
            def _():
                wait_out(b)

            nbuf = nbufs[b]
            fbuf = fbufs[b]
            ebuf = ebufs[b]
            obuf = obufs[b]

            def edge(i, _):
                obuf[i] = nbuf[i] + fbuf[i] + ebuf[i]
                return 0
            lax.fori_loop(0, C_CHUNK, edge, 0)

            pltpu.async_copy(obuf,
                             out_hbm.at[pl.ds(ebase + j * C_CHUNK, C_CHUNK)],
                             osems[b])

            @pl.when(jj < C_NCHUNK // 2 - 1)
            def _():
                start_in(j + 2, b)
        return 0
    lax.fori_loop(0, C_NCHUNK // 2, step, 0)

    for b in range(2):
        wait_out(b)


def _sc_c(no, fo, e3, idx):
    mesh = plsc.VectorSubcoreMesh(core_axis_name="c", subcore_axis_name="s")
    f = functools.partial(
        pl.kernel, _sc_c_body, mesh=mesh,
        compiler_params=pltpu.CompilerParams(use_tc_tiling_on_sc=False),
        out_type=jax.ShapeDtypeStruct((N_EDGES, DE), jnp.float32),
        scratch_types=(
            [pltpu.VMEM((C_NCHUNK, C_CHUNK), jnp.int32)] * 2
            + [pltpu.VMEM((C_CHUNK, DE), jnp.float32)] * 8
            + [pltpu.SemaphoreType.DMA] * 8
        ),
    )()
    return f(no, fo, e3, idx)


def kernel(node_emb, edge_emb, feature_emb, edge_index,
           P_src, b_P_src, P_dst, b_P_dst,
           Q_src_self, b_Q_src_self, Q_src_neigh, b_Q_src_neigh,
           Q_dst_self, b_Q_dst_self, Q_dst_neigh, b_Q_dst_neigh,
           W, b_W):
    psf, pse = P_src[:D], P_src[D:]
    pdf, pde = P_dst[:D], P_dst[D:]
    we, wn, wf = W[:DE], W[DE:DE + D], W[DE + D:]
    bps = b_P_src.reshape(1, D)
    bpd = b_P_dst.reshape(1, D)
    bw = b_W.reshape(1, DE)
    bn = (b_Q_src_self + b_Q_src_neigh).reshape(1, D)
    bf = (b_Q_dst_self + b_Q_dst_neigh).reshape(1, D)

    src = edge_index[0]
    dst = edge_index[1]
    gidxa = dst.reshape(16, AB_NSEG, AB_SEG, AB_CHUNK)
    gidxb = (src + N_NODES).reshape(16, AB_NSEG, AB_SEG, AB_CHUNK)
    sidx = edge_index.reshape(2, 16, AB_NSEG, AB_SEG, AB_CHUNK)
    idx_c = edge_index.reshape(64, C_NCHUNK, C_CHUNK)

    tab0, tab1 = _tc_nodes(feature_emb, node_emb, psf, pdf)
    est = _tc_edges(edge_emb, pse, pde, bps, bpd)
    e3 = _tc_e3(edge_emb, we, bw)

    acc0, acc1, cnt = _sc_ab(tab0.reshape(2 * N_NODES, DH),
                             tab1.reshape(2 * N_NODES, DH),
                             est, gidxa, gidxb, sidx)

    node_out, feat_out, no_w, fo_w = _tc_stage2(
        acc0, acc1, cnt, node_emb, feature_emb,
        Q_src_self, Q_src_neigh, Q_dst_self, Q_dst_neigh,
        wn, wf, bn, bf)

    edge_out = _sc_c(no_w, fo_w, e3, idx_c)
    return (node_out, edge_out, feat_out)
